# Initial kernel scaffold; baseline (speedup 1.0000x reference)
import jax
import jax.numpy as jnp
from jax.experimental import pallas as pl


def kernel(userA, userB, edge_label_index):
    raise NotImplementedError("write your pallas kernel here")


# speedup vs baseline: 2.5501x; 2.5501x over previous
import functools

import jax
import jax.numpy as jnp
from jax import lax
from jax.experimental import pallas as pl
from jax.experimental.pallas import tpu as pltpu
from jax.experimental.pallas import tpu_sc as plsc

_NC = 2
_NS = 16
_L = 16
_NW = _NC * _NS

_N = 10000
_D = 128
_D2 = _D // 2
_E = 320000
_EPW = _E // _NW
_CHUNK = 80
_NCHUNK = _EPW // _CHUNK
_GROUPS = _CHUNK // _L
_INV_TEMP = 10.0


def _normalize_body(a_ref, b_ref, na_ref, nb_ref):
    for src, dst in ((a_ref, na_ref), (b_ref, nb_ref)):
        x = src[...]
        norm = jnp.sqrt(jnp.sum(x * x, axis=-1, keepdims=True))
        y = x / jnp.maximum(norm, 1e-12)
        dst[...] = y.astype(jnp.bfloat16)


def _normalize(userA, userB):
    return pl.pallas_call(
        _normalize_body,
        out_shape=(
            jax.ShapeDtypeStruct((_N, _D), jnp.bfloat16),
            jax.ShapeDtypeStruct((_N, _D), jnp.bfloat16),
        ),
    )(userA, userB)


def _sc_body(tabA, tabB, idxA_hbm, idxB_hbm, out_hbm,
             idxA_v, idxB_v, rowsA_v, rowsB_v, out_v, sem):
    wid = lax.axis_index("s") * _NC + lax.axis_index("c")
    base = pl.multiple_of(wid * _EPW, 8)
    pltpu.sync_copy(idxA_hbm.at[pl.ds(base, _EPW)], idxA_v)
    pltpu.sync_copy(idxB_hbm.at[pl.ds(base, _EPW)], idxB_v)
    lanes = lax.iota(jnp.int32, _L)

    def chunk_body(c, _):
        off = pl.multiple_of(c * _CHUNK, 8)
        cpa = pltpu.async_copy(tabA.at[idxA_v.at[pl.ds(off, _CHUNK)]],
                               rowsA_v, sem)
        cpb = pltpu.async_copy(tabB.at[idxB_v.at[pl.ds(off, _CHUNK)]],
                               rowsB_v, sem)
        cpa.wait()
        cpb.wait()
        for g in range(_GROUPS):
            rowv = lanes + (g * _L)

            def fbody(f, carry):
                acc, colv = carry
                a = plsc.load_gather(rowsA_v, [rowv, colv])
                b = plsc.load_gather(rowsB_v, [rowv, colv])
                alo, ahi = plsc.unpack(plsc.bitcast(a, jnp.bfloat16),
                                       format=plsc.PackFormat.INTERLEAVED)
                blo, bhi = plsc.unpack(plsc.bitcast(b, jnp.bfloat16),
                                       format=plsc.PackFormat.INTERLEAVED)
                return acc + (alo * blo + ahi * bhi), colv + 1

            acc, _col = lax.fori_loop(
                0, _D2, fbody,
                (jnp.zeros((_L,), jnp.float32), jnp.zeros((_L,), jnp.int32)))
            out_v[pl.ds(g * _L, _L)] = acc * _INV_TEMP
        pltpu.sync_copy(out_v, out_hbm.at[pl.ds(base + off, _CHUNK)])
        return 0

    lax.fori_loop(0, _NCHUNK, chunk_body, 0)


_sc_call = functools.partial(
    pl.kernel,
    out_type=jax.ShapeDtypeStruct((_E,), jnp.float32),
    mesh=plsc.VectorSubcoreMesh(core_axis_name="c", subcore_axis_name="s"),
    compiler_params=pltpu.CompilerParams(needs_layout_passes=False,
                                         use_tc_tiling_on_sc=False),
    scratch_types=[
        pltpu.VMEM((_EPW,), jnp.int32),
        pltpu.VMEM((_EPW,), jnp.int32),
        pltpu.VMEM((_CHUNK, _D2), jnp.int32),
        pltpu.VMEM((_CHUNK, _D2), jnp.int32),
        pltpu.VMEM((_CHUNK,), jnp.float32),
        pltpu.SemaphoreType.DMA,
    ],
)(_sc_body)


def kernel(userA, userB, edge_label_index):
    na, nb = _normalize(userA, userB)
    tabA = lax.bitcast_convert_type(na.reshape(_N, _D2, 2), jnp.int32)
    tabB = lax.bitcast_convert_type(nb.reshape(_N, _D2, 2), jnp.int32)
    idx = edge_label_index.astype(jnp.int32)
    return _sc_call(tabA, tabB, idx[0], idx[1])


# speedup vs baseline: 2.6254x; 1.0295x over previous
import functools

import jax
import jax.numpy as jnp
from jax import lax
from jax.experimental import pallas as pl
from jax.experimental.pallas import tpu as pltpu
from jax.experimental.pallas import tpu_sc as plsc

_NC = 2
_NS = 16
_L = 16
_NW = _NC * _NS

_N = 10000
_D = 128
_D2 = _D // 2
_E = 320000
_EPW = _E // _NW
_CHUNK = 80
_NCHUNK = _EPW // _CHUNK
_GROUPS = _CHUNK // _L
_INV_TEMP = 10.0
_UNROLL = 8
_NACC = 4


def _normalize_body(a_ref, b_ref, na_ref, nb_ref):
    for src, dst in ((a_ref, na_ref), (b_ref, nb_ref)):
        x = src[...]
        norm = jnp.sqrt(jnp.sum(x * x, axis=-1, keepdims=True))
        y = x / jnp.maximum(norm, 1e-12)
        dst[...] = y.astype(jnp.bfloat16)


def _normalize(userA, userB):
    return pl.pallas_call(
        _normalize_body,
        out_shape=(
            jax.ShapeDtypeStruct((_N, _D), jnp.bfloat16),
            jax.ShapeDtypeStruct((_N, _D), jnp.bfloat16),
        ),
    )(userA, userB)


def _sc_body(tabA, tabB, idxA_hbm, idxB_hbm, out_hbm,
             idxA_v, idxB_v, rowsA_v, rowsB_v, out_v, sem):
    wid = lax.axis_index("s") * _NC + lax.axis_index("c")
    base = pl.multiple_of(wid * _EPW, 8)
    pltpu.sync_copy(idxA_hbm.at[pl.ds(base, _EPW)], idxA_v)
    pltpu.sync_copy(idxB_hbm.at[pl.ds(base, _EPW)], idxB_v)
    lanes = lax.iota(jnp.int32, _L)

    def chunk_body(c, _):
        off = pl.multiple_of(c * _CHUNK, 8)
        cpa = pltpu.async_copy(tabA.at[idxA_v.at[pl.ds(off, _CHUNK)]],
                               rowsA_v, sem)
        cpb = pltpu.async_copy(tabB.at[idxB_v.at[pl.ds(off, _CHUNK)]],
                               rowsB_v, sem)
        cpa.wait()
        cpb.wait()
        for g in range(_GROUPS):
            rowv = lanes + (g * _L)

            def fbody(t, carry):
                *accs, colv = carry
                accs = list(accs)
                for u in range(_UNROLL):
                    cu = colv + u
                    a = plsc.load_gather(rowsA_v, [rowv, cu])
                    b = plsc.load_gather(rowsB_v, [rowv, cu])
                    alo, ahi = plsc.unpack(plsc.bitcast(a, jnp.bfloat16),
                                           format=plsc.PackFormat.INTERLEAVED)
                    blo, bhi = plsc.unpack(plsc.bitcast(b, jnp.bfloat16),
                                           format=plsc.PackFormat.INTERLEAVED)
                    k = u % _NACC
                    accs[k] = accs[k] + (alo * blo + ahi * bhi)
                return (*accs, colv + _UNROLL)

            zero = jnp.zeros((_L,), jnp.float32)
            *accs, _col = lax.fori_loop(
                0, _D2 // _UNROLL, fbody,
                (zero,) * _NACC + (jnp.zeros((_L,), jnp.int32),))
            acc = (accs[0] + accs[1]) + (accs[2] + accs[3])
            out_v[pl.ds(g * _L, _L)] = acc * _INV_TEMP
        pltpu.sync_copy(out_v, out_hbm.at[pl.ds(base + off, _CHUNK)])
        return 0

    lax.fori_loop(0, _NCHUNK, chunk_body, 0)


_sc_call = functools.partial(
    pl.kernel,
    out_type=jax.ShapeDtypeStruct((_E,), jnp.float32),
    mesh=plsc.VectorSubcoreMesh(core_axis_name="c", subcore_axis_name="s"),
    compiler_params=pltpu.CompilerParams(needs_layout_passes=False,
                                         use_tc_tiling_on_sc=False),
    scratch_types=[
        pltpu.VMEM((_EPW,), jnp.int32),
        pltpu.VMEM((_EPW,), jnp.int32),
        pltpu.VMEM((_CHUNK, _D2), jnp.int32),
        pltpu.VMEM((_CHUNK, _D2), jnp.int32),
        pltpu.VMEM((_CHUNK,), jnp.float32),
        pltpu.SemaphoreType.DMA,
    ],
)(_sc_body)


def kernel(userA, userB, edge_label_index):
    na, nb = _normalize(userA, userB)
    tabA = lax.bitcast_convert_type(na.reshape(_N, _D2, 2), jnp.int32)
    tabB = lax.bitcast_convert_type(nb.reshape(_N, _D2, 2), jnp.int32)
    idx = edge_label_index.astype(jnp.int32)
    return _sc_call(tabA, tabB, idx[0], idx[1])


# speedup vs baseline: 3.1645x; 1.2053x over previous
import functools

import jax
import jax.numpy as jnp
from jax import lax
from jax.experimental import pallas as pl
from jax.experimental.pallas import tpu as pltpu
from jax.experimental.pallas import tpu_sc as plsc

_NC = 2
_NS = 16
_L = 16
_NW = _NC * _NS

_N = 10000
_D = 128
_D2 = _D // 2
_E = 320000
_EPW = _E // _NW
_CHUNK = 80
_NCHUNK = _EPW // _CHUNK
_GROUPS = _CHUNK // _L
_INV_TEMP = 10.0
_UNROLL = 8
_NACC = 4


def _normalize_body(a_ref, b_ref, na_ref, nb_ref):
    for src, dst in ((a_ref, na_ref), (b_ref, nb_ref)):
        x = src[...]
        norm = jnp.sqrt(jnp.sum(x * x, axis=-1, keepdims=True))
        y = x / jnp.maximum(norm, 1e-12)
        dst[...] = y.astype(jnp.bfloat16)


def _normalize(userA, userB):
    return pl.pallas_call(
        _normalize_body,
        out_shape=(
            jax.ShapeDtypeStruct((_N, _D), jnp.bfloat16),
            jax.ShapeDtypeStruct((_N, _D), jnp.bfloat16),
        ),
    )(userA, userB)


def _sc_body(tabA, tabB, idxA_hbm, idxB_hbm, out_hbm,
             idxA_v, idxB_v, rowsA0, rowsB0, rowsA1, rowsB1, out_v,
             sem0, sem1):
    wid = lax.axis_index("s") * _NC + lax.axis_index("c")
    base = pl.multiple_of(wid * _EPW, 8)
    pltpu.sync_copy(idxA_hbm.at[pl.ds(base, _EPW)], idxA_v)
    pltpu.sync_copy(idxB_hbm.at[pl.ds(base, _EPW)], idxB_v)
    lanes = lax.iota(jnp.int32, _L)
    bufs = ((rowsA0, rowsB0, sem0), (rowsA1, rowsB1, sem1))

    def issue(cc, ra, rb, sem):
        off = pl.multiple_of(cc * _CHUNK, 8)
        pltpu.async_copy(tabA.at[idxA_v.at[pl.ds(off, _CHUNK)]], ra, sem)
        pltpu.async_copy(tabB.at[idxB_v.at[pl.ds(off, _CHUNK)]], rb, sem)

    def drain(ra, rb, sem):
        pltpu.make_async_copy(tabA.at[idxA_v.at[pl.ds(0, _CHUNK)]],
                              ra, sem).wait()
        pltpu.make_async_copy(tabB.at[idxB_v.at[pl.ds(0, _CHUNK)]],
                              rb, sem).wait()

    def compute(c, rowsA_v, rowsB_v):
        off = pl.multiple_of(c * _CHUNK, 8)
        for g in range(_GROUPS):
            rowv = lanes + (g * _L)

            def fbody(t, carry):
                *accs, colv = carry
                accs = list(accs)
                for u in range(_UNROLL):
                    cu = colv + u
                    a = plsc.load_gather(rowsA_v, [rowv, cu])
                    b = plsc.load_gather(rowsB_v, [rowv, cu])
                    alo, ahi = plsc.unpack(plsc.bitcast(a, jnp.bfloat16),
                                           format=plsc.PackFormat.INTERLEAVED)
                    blo, bhi = plsc.unpack(plsc.bitcast(b, jnp.bfloat16),
                                           format=plsc.PackFormat.INTERLEAVED)
                    k = u % _NACC
                    accs[k] = accs[k] + (alo * blo + ahi * bhi)
                return (*accs, colv + _UNROLL)

            zero = jnp.zeros((_L,), jnp.float32)
            *accs, _col = lax.fori_loop(
                0, _D2 // _UNROLL, fbody,
                (zero,) * _NACC + (jnp.zeros((_L,), jnp.int32),))
            acc = (accs[0] + accs[1]) + (accs[2] + accs[3])
            out_v[pl.ds(g * _L, _L)] = acc * _INV_TEMP
        pltpu.sync_copy(out_v, out_hbm.at[pl.ds(base + off, _CHUNK)])

    issue(0, *bufs[0])
    issue(1, *bufs[1])

    def pair_body(p, _):
        c = p * 2
        for b in range(2):
            ra, rb, sem = bufs[b]
            cc = c + b

            @pl.when(cc < _NCHUNK)
            def _process():
                drain(ra, rb, sem)
                compute(cc, ra, rb)

                @pl.when(cc + 2 < _NCHUNK)
                def _prefetch():
                    issue(cc + 2, ra, rb, sem)

        return 0

    lax.fori_loop(0, (_NCHUNK + 1) // 2, pair_body, 0)


_sc_call = functools.partial(
    pl.kernel,
    out_type=jax.ShapeDtypeStruct((_E,), jnp.float32),
    mesh=plsc.VectorSubcoreMesh(core_axis_name="c", subcore_axis_name="s"),
    compiler_params=pltpu.CompilerParams(needs_layout_passes=False,
                                         use_tc_tiling_on_sc=False),
    scratch_types=[
        pltpu.VMEM((_EPW,), jnp.int32),
        pltpu.VMEM((_EPW,), jnp.int32),
        pltpu.VMEM((_CHUNK, _D2), jnp.int32),
        pltpu.VMEM((_CHUNK, _D2), jnp.int32),
        pltpu.VMEM((_CHUNK, _D2), jnp.int32),
        pltpu.VMEM((_CHUNK, _D2), jnp.int32),
        pltpu.VMEM((_CHUNK,), jnp.float32),
        pltpu.SemaphoreType.DMA,
        pltpu.SemaphoreType.DMA,
    ],
)(_sc_body)


def kernel(userA, userB, edge_label_index):
    na, nb = _normalize(userA, userB)
    tabA = lax.bitcast_convert_type(na.reshape(_N, _D2, 2), jnp.int32)
    tabB = lax.bitcast_convert_type(nb.reshape(_N, _D2, 2), jnp.int32)
    idx = edge_label_index.astype(jnp.int32)
    return _sc_call(tabA, tabB, idx[0], idx[1])


# speedup vs baseline: 10.2158x; 3.2282x over previous
import functools

import jax
import jax.numpy as jnp
from jax import lax
from jax.experimental import pallas as pl
from jax.experimental.pallas import tpu as pltpu
from jax.experimental.pallas import tpu_sc as plsc

_NC = 2
_NS = 16
_L = 16
_NW = _NC * _NS

_N = 10000
_D = 128
_D2 = _D // 2
_E = 320000
_EPW = _E // _NW
_CHUNK = 80
_NCHUNK = _EPW // _CHUNK
_GROUPS = _CHUNK // _L
_INV_TEMP = 10.0
_EUNROLL = 4


def _normalize_body(a_ref, b_ref, na_ref, nb_ref):
    for src, dst in ((a_ref, na_ref), (b_ref, nb_ref)):
        x = src[...]
        norm = jnp.sqrt(jnp.sum(x * x, axis=-1, keepdims=True))
        y = x / jnp.maximum(norm, 1e-12)
        dst[...] = y.astype(jnp.bfloat16)


def _normalize(userA, userB):
    return pl.pallas_call(
        _normalize_body,
        out_shape=(
            jax.ShapeDtypeStruct((_N, _D), jnp.bfloat16),
            jax.ShapeDtypeStruct((_N, _D), jnp.bfloat16),
        ),
    )(userA, userB)


def _sc_body(tabA, tabB, idxA_hbm, idxB_hbm, out_hbm,
             idxA_v, idxB_v, rowsA0, rowsB0, rowsA1, rowsB1, out_v,
             sem0, sem1):
    wid = lax.axis_index("s") * _NC + lax.axis_index("c")
    base = pl.multiple_of(wid * _EPW, 8)
    pltpu.sync_copy(idxA_hbm.at[pl.ds(base, _EPW)], idxA_v)
    pltpu.sync_copy(idxB_hbm.at[pl.ds(base, _EPW)], idxB_v)
    lanes = lax.iota(jnp.int32, _L)
    bufs = ((rowsA0, rowsB0, sem0), (rowsA1, rowsB1, sem1))

    def issue(cc, ra, rb, sem):
        off = pl.multiple_of(cc * _CHUNK, 8)
        pltpu.async_copy(tabA.at[idxA_v.at[pl.ds(off, _CHUNK)]], ra, sem)
        pltpu.async_copy(tabB.at[idxB_v.at[pl.ds(off, _CHUNK)]], rb, sem)

    def drain(ra, rb, sem):
        pltpu.make_async_copy(tabA.at[idxA_v.at[pl.ds(0, _CHUNK)]],
                              ra, sem).wait()
        pltpu.make_async_copy(tabB.at[idxB_v.at[pl.ds(0, _CHUNK)]],
                              rb, sem).wait()

    def compute(c, rowsA_v, rowsB_v):
        off = pl.multiple_of(c * _CHUNK, 8)

        def ebody(t, _):
            res = jnp.zeros((_L,), jnp.float32)
            for u in range(_L):
                e = t * _L + u
                acc0 = jnp.zeros((_L,), jnp.float32)
                acc1 = jnp.zeros((_L,), jnp.float32)
                for q in range(_D2 // _L):
                    a = plsc.bitcast(rowsA_v[e, pl.ds(q * _L, _L)],
                                     jnp.bfloat16)
                    b = plsc.bitcast(rowsB_v[e, pl.ds(q * _L, _L)],
                                     jnp.bfloat16)
                    plo, phi = plsc.unpack(a * b,
                                           format=plsc.PackFormat.INTERLEAVED)
                    acc0 = acc0 + plo
                    acc1 = acc1 + phi
                res = jnp.where(lanes == u, jnp.sum(acc0 + acc1), res)
            out_v[pl.ds(pl.multiple_of(t * _L, 8), _L)] = res * _INV_TEMP
            return 0

        lax.fori_loop(0, _CHUNK // _L, ebody, 0)
        pltpu.sync_copy(out_v, out_hbm.at[pl.ds(base + off, _CHUNK)])

    issue(0, *bufs[0])
    issue(1, *bufs[1])

    def pair_body(p, _):
        c = p * 2
        for b in range(2):
            ra, rb, sem = bufs[b]
            cc = c + b

            @pl.when(cc < _NCHUNK)
            def _process():
                drain(ra, rb, sem)
                compute(cc, ra, rb)

                @pl.when(cc + 2 < _NCHUNK)
                def _prefetch():
                    issue(cc + 2, ra, rb, sem)

        return 0

    lax.fori_loop(0, (_NCHUNK + 1) // 2, pair_body, 0)


_sc_call = functools.partial(
    pl.kernel,
    out_type=jax.ShapeDtypeStruct((_E,), jnp.float32),
    mesh=plsc.VectorSubcoreMesh(core_axis_name="c", subcore_axis_name="s"),
    compiler_params=pltpu.CompilerParams(needs_layout_passes=False,
                                         use_tc_tiling_on_sc=False),
    scratch_types=[
        pltpu.VMEM((_EPW,), jnp.int32),
        pltpu.VMEM((_EPW,), jnp.int32),
        pltpu.VMEM((_CHUNK, _D2), jnp.int32),
        pltpu.VMEM((_CHUNK, _D2), jnp.int32),
        pltpu.VMEM((_CHUNK, _D2), jnp.int32),
        pltpu.VMEM((_CHUNK, _D2), jnp.int32),
        pltpu.VMEM((_CHUNK,), jnp.float32),
        pltpu.SemaphoreType.DMA,
        pltpu.SemaphoreType.DMA,
    ],
)(_sc_body)


def kernel(userA, userB, edge_label_index):
    na, nb = _normalize(userA, userB)
    tabA = lax.bitcast_convert_type(na.reshape(_N, _D2, 2), jnp.int32)
    tabB = lax.bitcast_convert_type(nb.reshape(_N, _D2, 2), jnp.int32)
    idx = edge_label_index.astype(jnp.int32)
    return _sc_call(tabA, tabB, idx[0], idx[1])


# speedup vs baseline: 14.4896x; 1.4184x over previous
import functools

import jax
import jax.numpy as jnp
from jax import lax
from jax.experimental import pallas as pl
from jax.experimental.pallas import tpu as pltpu
from jax.experimental.pallas import tpu_sc as plsc

_NC = 2
_NS = 16
_L = 16
_NW = _NC * _NS

_N = 10000
_D = 128
_D2 = _D // 2
_E = 320000
_EPW = _E // _NW
_CHUNK = 80
_NCHUNK = _EPW // _CHUNK
_GROUPS = _CHUNK // _L
_INV_TEMP = 10.0
_EUNROLL = 4


def _normalize_body(a_ref, b_ref, pa_ref, pb_ref):
    for src, dst in ((a_ref, pa_ref), (b_ref, pb_ref)):
        x = src[...]
        norm = jnp.sqrt(jnp.sum(x * x, axis=-1, keepdims=True))
        y = (x / jnp.maximum(norm, 1e-12)).astype(jnp.bfloat16)
        lo = lax.bitcast_convert_type(y[:, :_D2], jnp.uint16).astype(jnp.uint32)
        hi = lax.bitcast_convert_type(y[:, _D2:], jnp.uint16).astype(jnp.uint32)
        dst[...] = lo | (hi << 16)


def _normalize(userA, userB):
    return pl.pallas_call(
        _normalize_body,
        out_shape=(
            jax.ShapeDtypeStruct((_N, _D2), jnp.uint32),
            jax.ShapeDtypeStruct((_N, _D2), jnp.uint32),
        ),
    )(userA, userB)


def _sc_body(tabA, tabB, idx_hbm, out_hbm,
             idxA_v, idxB_v, rowsA0, rowsB0, rowsA1, rowsB1, out_v,
             sem0, sem1):
    wid = lax.axis_index("s") * _NC + lax.axis_index("c")
    base = pl.multiple_of(wid * _EPW, 8)
    pltpu.sync_copy(idx_hbm.at[0, pl.ds(base, _EPW)], idxA_v)
    pltpu.sync_copy(idx_hbm.at[1, pl.ds(base, _EPW)], idxB_v)
    lanes = lax.iota(jnp.int32, _L)
    bufs = ((rowsA0, rowsB0, sem0), (rowsA1, rowsB1, sem1))

    def issue(cc, ra, rb, sem):
        off = pl.multiple_of(cc * _CHUNK, 8)
        pltpu.async_copy(tabA.at[idxA_v.at[pl.ds(off, _CHUNK)]], ra, sem)
        pltpu.async_copy(tabB.at[idxB_v.at[pl.ds(off, _CHUNK)]], rb, sem)

    def drain(ra, rb, sem):
        pltpu.make_async_copy(tabA.at[idxA_v.at[pl.ds(0, _CHUNK)]],
                              ra, sem).wait()
        pltpu.make_async_copy(tabB.at[idxB_v.at[pl.ds(0, _CHUNK)]],
                              rb, sem).wait()

    def compute(c, rowsA_v, rowsB_v):
        off = pl.multiple_of(c * _CHUNK, 8)

        def ebody(t, _):
            res = jnp.zeros((_L,), jnp.float32)
            for u in range(_L):
                e = t * _L + u
                acc0 = jnp.zeros((_L,), jnp.float32)
                acc1 = jnp.zeros((_L,), jnp.float32)
                for q in range(_D2 // _L):
                    a = plsc.bitcast(rowsA_v[e, pl.ds(q * _L, _L)],
                                     jnp.bfloat16)
                    b = plsc.bitcast(rowsB_v[e, pl.ds(q * _L, _L)],
                                     jnp.bfloat16)
                    plo, phi = plsc.unpack(a * b,
                                           format=plsc.PackFormat.INTERLEAVED)
                    acc0 = acc0 + plo
                    acc1 = acc1 + phi
                res = jnp.where(lanes == u, jnp.sum(acc0 + acc1), res)
            out_v[pl.ds(pl.multiple_of(t * _L, 8), _L)] = res * _INV_TEMP
            return 0

        lax.fori_loop(0, _CHUNK // _L, ebody, 0)
        pltpu.sync_copy(out_v, out_hbm.at[pl.ds(base + off, _CHUNK)])

    issue(0, *bufs[0])
    issue(1, *bufs[1])

    def pair_body(p, _):
        c = p * 2
        for b in range(2):
            ra, rb, sem = bufs[b]
            cc = c + b

            @pl.when(cc < _NCHUNK)
            def _process():
                drain(ra, rb, sem)
                compute(cc, ra, rb)

                @pl.when(cc + 2 < _NCHUNK)
                def _prefetch():
                    issue(cc + 2, ra, rb, sem)

        return 0

    lax.fori_loop(0, (_NCHUNK + 1) // 2, pair_body, 0)


_sc_call = functools.partial(
    pl.kernel,
    out_type=jax.ShapeDtypeStruct((_E,), jnp.float32),
    mesh=plsc.VectorSubcoreMesh(core_axis_name="c", subcore_axis_name="s"),
    compiler_params=pltpu.CompilerParams(needs_layout_passes=False,
                                         use_tc_tiling_on_sc=False),
    scratch_types=[
        pltpu.VMEM((_EPW,), jnp.int32),
        pltpu.VMEM((_EPW,), jnp.int32),
        pltpu.VMEM((_CHUNK, _D2), jnp.uint32),
        pltpu.VMEM((_CHUNK, _D2), jnp.uint32),
        pltpu.VMEM((_CHUNK, _D2), jnp.uint32),
        pltpu.VMEM((_CHUNK, _D2), jnp.uint32),
        pltpu.VMEM((_CHUNK,), jnp.float32),
        pltpu.SemaphoreType.DMA,
        pltpu.SemaphoreType.DMA,
    ],
)(_sc_body)


def kernel(userA, userB, edge_label_index):
    tabA, tabB = _normalize(userA, userB)
    idx = edge_label_index.astype(jnp.int32)
    return _sc_call(tabA, tabB, idx)


# speedup vs baseline: 16.4269x; 1.1337x over previous
import functools

import jax
import jax.numpy as jnp
from jax import lax
from jax.experimental import pallas as pl
from jax.experimental.pallas import tpu as pltpu
from jax.experimental.pallas import tpu_sc as plsc

_NC = 2
_NS = 16
_L = 16
_NW = _NC * _NS

_N = 10000
_D = 128
_D2 = _D // 2
_E = 320000
_EPW = _E // _NW
_CHUNK = 128
_NCHUNK = -(-_EPW // _CHUNK)
_LAST_OFF = _EPW - _CHUNK
_INV_TEMP = 10.0


def _normalize_body(a_ref, b_ref, pa_ref, pb_ref):
    for src, dst in ((a_ref, pa_ref), (b_ref, pb_ref)):
        x = src[...]
        norm = jnp.sqrt(jnp.sum(x * x, axis=-1, keepdims=True))
        y = (x / jnp.maximum(norm, 1e-12)).astype(jnp.bfloat16)
        lo = lax.bitcast_convert_type(y[:, :_D2], jnp.uint16).astype(jnp.uint32)
        hi = lax.bitcast_convert_type(y[:, _D2:], jnp.uint16).astype(jnp.uint32)
        dst[...] = lo | (hi << 16)


def _normalize(userA, userB):
    return pl.pallas_call(
        _normalize_body,
        out_shape=(
            jax.ShapeDtypeStruct((_N, _D2), jnp.uint32),
            jax.ShapeDtypeStruct((_N, _D2), jnp.uint32),
        ),
    )(userA, userB)


def _sc_body(tabA, tabB, idx_hbm, out_hbm,
             idxA_v, idxB_v, rowsA0, rowsB0, rowsA1, rowsB1, out_v,
             sem0, sem1):
    wid = lax.axis_index("s") * _NC + lax.axis_index("c")
    base = pl.multiple_of(wid * _EPW, 8)
    pltpu.sync_copy(idx_hbm.at[0, pl.ds(base, _EPW)], idxA_v)
    pltpu.sync_copy(idx_hbm.at[1, pl.ds(base, _EPW)], idxB_v)
    lanes = lax.iota(jnp.int32, _L)
    bufs = ((rowsA0, rowsB0, sem0), (rowsA1, rowsB1, sem1))

    def chunk_off(cc):
        return pl.multiple_of(jnp.minimum(cc * _CHUNK, _LAST_OFF), 8)

    def issue(cc, ra, rb, sem):
        off = chunk_off(cc)
        pltpu.async_copy(tabA.at[idxA_v.at[pl.ds(off, _CHUNK)]], ra, sem)
        pltpu.async_copy(tabB.at[idxB_v.at[pl.ds(off, _CHUNK)]], rb, sem)

    def drain(ra, rb, sem):
        pltpu.make_async_copy(tabA.at[idxA_v.at[pl.ds(0, _CHUNK)]],
                              ra, sem).wait()
        pltpu.make_async_copy(tabB.at[idxB_v.at[pl.ds(0, _CHUNK)]],
                              rb, sem).wait()

    def compute(c, rowsA_v, rowsB_v):
        off = chunk_off(c)

        def ebody(t, _):
            res = jnp.zeros((_L,), jnp.float32)
            for u in range(_L):
                e = t * _L + u
                ps = []
                for q in range(_D2 // _L):
                    a = plsc.bitcast(rowsA_v[e, pl.ds(q * _L, _L)],
                                     jnp.bfloat16)
                    b = plsc.bitcast(rowsB_v[e, pl.ds(q * _L, _L)],
                                     jnp.bfloat16)
                    ps.append(a * b)
                s = (ps[0] + ps[1]) + (ps[2] + ps[3])
                plo, phi = plsc.unpack(s, format=plsc.PackFormat.INTERLEAVED)
                res = jnp.where(lanes == u, jnp.sum(plo + phi), res)
            out_v[pl.ds(pl.multiple_of(t * _L, 8), _L)] = res * _INV_TEMP
            return 0

        lax.fori_loop(0, _CHUNK // _L, ebody, 0)
        pltpu.sync_copy(out_v, out_hbm.at[pl.ds(base + off, _CHUNK)])

    issue(0, *bufs[0])
    issue(1, *bufs[1])

    def pair_body(p, _):
        c = p * 2
        for b in range(2):
            ra, rb, sem = bufs[b]
            cc = c + b

            @pl.when(cc < _NCHUNK)
            def _process():
                drain(ra, rb, sem)
                compute(cc, ra, rb)

                @pl.when(cc + 2 < _NCHUNK)
                def _prefetch():
                    issue(cc + 2, ra, rb, sem)

        return 0

    lax.fori_loop(0, (_NCHUNK + 1) // 2, pair_body, 0)


_sc_call = functools.partial(
    pl.kernel,
    out_type=jax.ShapeDtypeStruct((_E,), jnp.float32),
    mesh=plsc.VectorSubcoreMesh(core_axis_name="c", subcore_axis_name="s"),
    compiler_params=pltpu.CompilerParams(needs_layout_passes=False,
                                         use_tc_tiling_on_sc=False),
    scratch_types=[
        pltpu.VMEM((_EPW,), jnp.int32),
        pltpu.VMEM((_EPW,), jnp.int32),
        pltpu.VMEM((_CHUNK, _D2), jnp.uint32),
        pltpu.VMEM((_CHUNK, _D2), jnp.uint32),
        pltpu.VMEM((_CHUNK, _D2), jnp.uint32),
        pltpu.VMEM((_CHUNK, _D2), jnp.uint32),
        pltpu.VMEM((_CHUNK,), jnp.float32),
        pltpu.SemaphoreType.DMA,
        pltpu.SemaphoreType.DMA,
    ],
)(_sc_body)


def kernel(userA, userB, edge_label_index):
    tabA, tabB = _normalize(userA, userB)
    idx = edge_label_index.astype(jnp.int32)
    return _sc_call(tabA, tabB, idx)
